# Initial kernel scaffold; baseline (speedup 1.0000x reference)
import jax
import jax.numpy as jnp
from jax.experimental import pallas as pl


def kernel(x, W1, b1, W2):
    raise NotImplementedError("write your pallas kernel here")


# speedup vs baseline: 1.9814x; 1.9814x over previous
import functools

import jax
import jax.numpy as jnp
from jax import lax
from jax.experimental import pallas as pl
from jax.experimental.pallas import tpu as pltpu
from jax.experimental.pallas import tpu_sc as plsc

_TBLK = 256
_K_EASY = 409
_K_HARD = 102
_KE_PAD = 416
_KH_PAD = 104
_IDX_W = 2 * _KE_PAD + 2 * _KH_PAD
_ROWS = 4 * (_K_EASY * 2 + _K_HARD * 2)
_ROWS_PAD = 4096


def _conv1_body(xp_ref, w_ref, b1_ref, e_ref):
    t = pl.program_id(1)
    base = t * _TBLK
    xs = xp_ref[0, pl.ds(base, _TBLK + 8), :]
    acc = jnp.dot(xs[0:_TBLK], w_ref[0], preferred_element_type=jnp.float32)
    acc = acc + jnp.dot(xs[1:_TBLK + 1], w_ref[1],
                        preferred_element_type=jnp.float32)
    acc = acc + jnp.dot(xs[2:_TBLK + 2], w_ref[2],
                        preferred_element_type=jnp.float32)
    e_ref[0] = jnp.maximum(acc + b1_ref[...], 0.0)


def _pallas_conv1(x, W1, b1):
    B, T, D = x.shape
    O = W1.shape[0]
    xp = jnp.pad(x, ((0, 0), (1, 7), (0, 0)))
    wt = jnp.transpose(W1, (2, 1, 0))
    return pl.pallas_call(
        _conv1_body,
        grid=(B, T // _TBLK),
        in_specs=[
            pl.BlockSpec((1, T + 8, D), lambda b, t: (b, 0, 0)),
            pl.BlockSpec((3, D, O), lambda b, t: (0, 0, 0)),
            pl.BlockSpec((1, O), lambda b, t: (0, 0)),
        ],
        out_specs=pl.BlockSpec((1, _TBLK, O), lambda b, t: (b, t, 0)),
        out_shape=jax.ShapeDtypeStruct((B, T, O), jnp.float32),
    )(xp, wt, b1[None, :])


def _rank_row(key_row, key_col, jlt_blocks):
    T = key_row.shape[1]
    acc = jnp.zeros((1, T), jnp.float32)
    for c in range(T // _TBLK):
        kc = key_col[c * _TBLK:(c + 1) * _TBLK, :]
        gt = kc > key_row
        eqilt = (kc == key_row) & jlt_blocks[c]
        acc = acc + jnp.sum(jnp.where(gt | eqilt, 1.0, 0.0),
                            axis=0, keepdims=True)
    return acc


def _shift1(v, o, axis):
    if o == 0:
        return v
    n = v.shape[axis]
    if axis == 1:
        z = jnp.zeros((1, abs(o)), v.dtype)
        if o > 0:
            return jnp.concatenate([v[:, o:], z], axis=1)
        return jnp.concatenate([z, v[:, :n + o]], axis=1)
    z = jnp.zeros((abs(o), 1), v.dtype)
    if o > 0:
        return jnp.concatenate([v[o:, :], z], axis=0)
    return jnp.concatenate([z, v[:n + o, :]], axis=0)


def _minmax_window(v, offs, axis, is_min):
    out = _shift1(v, offs[0], axis)
    for o in offs[1:]:
        s = _shift1(v, o, axis)
        out = jnp.minimum(out, s) if is_min else jnp.maximum(out, s)
    return out


def _masks(bin_v, axis):
    ero_M = _minmax_window(bin_v, [-3, -2, -1, 0, 1, 2], axis, True)
    ero_m = _minmax_window(bin_v, [-1, 0, 1], axis, True)
    dil_m = _minmax_window(bin_v, [1, 0, -1], axis, False)
    dil_M = _minmax_window(bin_v, [3, 2, 1, 0, -1, -2], axis, False)
    return ero_m - ero_M, dil_M - dil_m


def _select_body(act_row_ref, act_col_ref, casT_ref, vs_ref, idx_ref):
    b = pl.program_id(0)
    T = act_row_ref.shape[2]
    a_row = act_row_ref[0]
    a_col = act_col_ref[0]

    jlt_blocks = []
    j_iota = lax.broadcasted_iota(jnp.int32, (1, T), 1)
    i_iota = lax.broadcasted_iota(jnp.int32, (_TBLK, 1), 0)
    for c in range(T // _TBLK):
        jlt_blocks.append((i_iota + c * _TBLK) < j_iota)

    rank1 = _rank_row(a_row, a_col, jlt_blocks)
    maxa = jnp.max(a_row)
    s2_row = maxa - a_row
    s2_col = maxa - a_col
    rank2 = _rank_row(s2_row, s2_col, jlt_blocks)

    v_lo = jnp.sum(jnp.where(rank1 == 1024.0, a_row, 0.0), axis=1,
                   keepdims=True)
    v_hi = jnp.sum(jnp.where(rank1 == 1023.0, a_row, 0.0), axis=1,
                   keepdims=True)
    med = (v_lo + v_hi) * 0.5

    bin_row = jnp.where(a_row > med, 1.0, 0.0)
    bin_col = jnp.where(a_col > med, 1.0, 0.0)
    inner_row, outer_row = _masks(bin_row, 1)
    inner_col, outer_col = _masks(bin_col, 0)

    rank3 = _rank_row(a_row * inner_row, a_col * inner_col, jlt_blocks)
    rank4 = _rank_row(a_row * outer_row, a_col * outer_col, jlt_blocks)

    ji = lax.broadcasted_iota(jnp.int32, (1, T), 1)
    boff = b * T
    for rank, kpad, off in ((rank1, _KE_PAD, 0),
                            (rank2, _KE_PAD, _KE_PAD),
                            (rank3, _KH_PAD, 2 * _KE_PAD),
                            (rank4, _KH_PAD, 2 * _KE_PAD + _KH_PAD)):
        r_col = lax.broadcasted_iota(jnp.int32, (kpad, 1), 0).astype(jnp.float32)
        eq = rank == r_col
        v = jnp.sum(jnp.where(eq, ji, 0), axis=1, keepdims=True)
        idx_ref[0, off:off + kpad, :] = v + boff

    vals = casT_ref[0]
    keys = lax.bitcast_convert_type(vals, jnp.int32)
    kf = jnp.float32(_K_EASY)

    def it(i, X):
        bit = 30 - i
        cand = X | lax.shift_left(jnp.int32(1), bit)
        cnt = jnp.sum(jnp.where(keys >= cand, 1.0, 0.0), axis=1,
                      keepdims=True)
        return jnp.where(cnt >= kf, cand, X)

    X = lax.fori_loop(0, 31, it, jnp.zeros((vals.shape[0], 1), jnp.int32))
    tval = lax.bitcast_convert_type(X, jnp.float32)
    gt = keys > X
    cnt_gt = jnp.sum(jnp.where(gt, 1.0, 0.0), axis=1, keepdims=True)
    sum_gt = jnp.sum(jnp.where(gt, vals, 0.0), axis=1, keepdims=True)
    mean_topk = (sum_gt + (kf - cnt_gt) * tval) / kf

    valid = lax.broadcasted_iota(jnp.int32, (vals.shape[0], 1), 0) < 20
    m = jnp.where(valid, mean_topk, -jnp.inf)
    mx = jnp.max(m, axis=0, keepdims=True)
    e = jnp.where(valid, jnp.exp(m - mx), 0.0)
    vs_ref[0] = e / jnp.sum(e, axis=0, keepdims=True)


def _pallas_select(actionness, casT):
    B, T = actionness.shape
    C = casT.shape[1]
    vs, idx = pl.pallas_call(
        _select_body,
        grid=(B,),
        in_specs=[
            pl.BlockSpec((1, 1, T), lambda b: (b, 0, 0)),
            pl.BlockSpec((1, T, 1), lambda b: (b, 0, 0)),
            pl.BlockSpec((1, C, T), lambda b: (b, 0, 0)),
        ],
        out_specs=[
            pl.BlockSpec((1, C, 1), lambda b: (b, 0, 0)),
            pl.BlockSpec((1, _IDX_W, 1), lambda b: (b, 0, 0)),
        ],
        out_shape=[
            jax.ShapeDtypeStruct((B, C, 1), jnp.float32),
            jax.ShapeDtypeStruct((B, _IDX_W, 1), jnp.int32),
        ],
    )(actionness[:, None, :], actionness[:, :, None], casT)
    return vs[:, :20, 0], idx[:, :, 0]


def _sc_gather(table, idx):
    NW = 32
    CH = 16
    n, d = idx.shape[0], table.shape[1]
    per_w = n // NW
    mesh = plsc.VectorSubcoreMesh(core_axis_name="c", subcore_axis_name="s")

    @functools.partial(
        pl.kernel,
        out_type=jax.ShapeDtypeStruct((n, d), jnp.float32),
        mesh=mesh,
        scratch_types=[
            pltpu.VMEM((CH,), jnp.int32),
            pltpu.VMEM((CH, d), jnp.float32),
            pltpu.SemaphoreType.DMA,
        ],
    )
    def k(table_hbm, idx_hbm, out_hbm, idx_v, rows_v, sem):
        wid = lax.axis_index("s") * 2 + lax.axis_index("c")
        base = wid * per_w
        for c in range(per_w // CH):
            off = base + c * CH
            pltpu.sync_copy(idx_hbm.at[pl.ds(off, CH)], idx_v)
            pltpu.async_copy(table_hbm.at[idx_v], rows_v, sem).wait()
            pltpu.sync_copy(rows_v, out_hbm.at[pl.ds(off, CH)])

    return k(table, idx)


def kernel(x, W1, b1, W2):
    B, T, D = x.shape
    O = W1.shape[0]

    out = jnp.transpose(x, (0, 2, 1))
    out = lax.conv_general_dilated(out, W1, (1,), [(1, 1)],
                                   dimension_numbers=('NCH', 'OIH', 'NCH'))
    out = jax.nn.relu(out + b1[None, :, None])
    embeddings_x = jnp.transpose(out, (0, 2, 1))
    out2 = lax.conv_general_dilated(out, W2, (1,), [(0, 0)],
                                    dimension_numbers=('NCH', 'OIH', 'NCH'))
    out2 = jax.nn.relu(out2)
    cas = jnp.transpose(out2, (0, 2, 1))
    actionness = cas.sum(axis=2)
    del embeddings_x

    E = _pallas_conv1(x, W1, b1)

    casT = jnp.pad(jnp.transpose(cas, (0, 2, 1)),
                   ((0, 0), (0, 4), (0, 0)))

    def _run(ops):
        act, cT, table = ops
        video_scores, idx2d = _pallas_select(act, cT)
        ea = idx2d[:, 0:_K_EASY].reshape(-1)
        eb = idx2d[:, _KE_PAD:_KE_PAD + _K_EASY].reshape(-1)
        ha = idx2d[:, 2 * _KE_PAD:2 * _KE_PAD + _K_HARD].reshape(-1)
        hb = idx2d[:, 2 * _KE_PAD + _KH_PAD:
                   2 * _KE_PAD + _KH_PAD + _K_HARD].reshape(-1)
        idx_flat = jnp.concatenate(
            [ea, eb, ha, hb, jnp.zeros((_ROWS_PAD - _ROWS,), jnp.int32)])
        return video_scores, _sc_gather(table, idx_flat)

    def _skip(ops):
        return (jnp.zeros((B, 20), jnp.float32),
                jnp.zeros((_ROWS_PAD, O), jnp.float32))

    pred = actionness[0, 0] == actionness[0, 0]
    video_scores, g = lax.cond(pred, _run, _skip,
                               (actionness, casT, E.reshape(B * T, O)))

    n_e = B * _K_EASY
    n_h = B * _K_HARD
    easy_act = g[0:n_e].reshape(B, _K_EASY, O)
    easy_bkg = g[n_e:2 * n_e].reshape(B, _K_EASY, O)
    hard_act = g[2 * n_e:2 * n_e + n_h].reshape(B, _K_HARD, O)
    hard_bkg = g[2 * n_e + n_h:2 * n_e + 2 * n_h].reshape(B, _K_HARD, O)

    return (video_scores, easy_act, easy_bkg, hard_act, hard_bkg,
            actionness, cas)
